# Initial kernel scaffold; baseline (speedup 1.0000x reference)
import jax
import jax.numpy as jnp
from jax.experimental import pallas as pl


def kernel(x, edge_index, W_src, W_tgt, W_ih, W_hh, b_ih, b_hh):
    raise NotImplementedError("write your pallas kernel here")


# speedup vs baseline: 1.4956x; 1.4956x over previous
import functools

import jax
import jax.numpy as jnp
from jax.experimental import pallas as pl
from jax.experimental.pallas import tpu as pltpu

N = 10000
E = 160000
D_IN = 256
D_OUT = 512
G = 4 * D_OUT

_NB = 10
_BN = N // _NB
_EB = 160
_BE = E // _EB


def _pre_kernel(x_ref, wsrc_ref, wih_ref, wtgt_ref, sx_ref, wcomb_ref):
    x = x_ref[...]
    sx_ref[...] = jax.lax.dot_general(
        x, wsrc_ref[...], (((1,), (1,)), ((), ())),
        preferred_element_type=jnp.float32)
    @pl.when(pl.program_id(0) == 0)
    def _():
        wcomb_ref[...] = jax.lax.dot_general(
            wih_ref[...], wtgt_ref[...], (((1,), (0,)), ((), ())),
            preferred_element_type=jnp.float32)


def _pre(x, W_src, W_ih, W_tgt):
    return pl.pallas_call(
        _pre_kernel,
        grid=(_NB,),
        in_specs=[
            pl.BlockSpec((_BN, D_IN), lambda i: (i, 0)),
            pl.BlockSpec((D_OUT, D_IN), lambda i: (0, 0)),
            pl.BlockSpec((G, D_OUT), lambda i: (0, 0)),
            pl.BlockSpec((D_OUT, D_IN), lambda i: (0, 0)),
        ],
        out_specs=[
            pl.BlockSpec((_BN, D_OUT), lambda i: (i, 0)),
            pl.BlockSpec((G, D_IN), lambda i: (0, 0)),
        ],
        out_shape=[
            jax.ShapeDtypeStruct((N, D_OUT), jnp.float32),
            jax.ShapeDtypeStruct((G, D_IN), jnp.float32),
        ],
    )(x, W_src, W_ih, W_tgt)


def _agg_kernel(src_ref, dst_ref, sx_ref, out_ref):
    step = pl.program_id(0)

    @pl.when(step == 0)
    def _():
        out_ref[...] = jnp.full((N, D_OUT), -jnp.inf, dtype=jnp.float32)

    def body(i, carry):
        s = src_ref[0, 0, i]
        d = dst_ref[0, 0, i]
        row = sx_ref[pl.ds(s, 1), :]
        cur = out_ref[pl.ds(d, 1), :]
        out_ref[pl.ds(d, 1), :] = jnp.maximum(cur, row)
        return carry

    jax.lax.fori_loop(0, _BE, body, 0)

    @pl.when(step == _EB - 1)
    def _():
        a = out_ref[...]
        out_ref[...] = jnp.where(a == -jnp.inf, 0.0, a)


def _aggregate(src2d, dst2d, source_x):
    return pl.pallas_call(
        _agg_kernel,
        grid=(_EB,),
        in_specs=[
            pl.BlockSpec((1, 1, _BE), lambda i: (i, 0, 0), memory_space=pltpu.SMEM),
            pl.BlockSpec((1, 1, _BE), lambda i: (i, 0, 0), memory_space=pltpu.SMEM),
            pl.BlockSpec((N, D_OUT), lambda i: (0, 0)),
        ],
        out_specs=pl.BlockSpec((N, D_OUT), lambda i: (0, 0)),
        out_shape=jax.ShapeDtypeStruct((N, D_OUT), jnp.float32),
    )(src2d, dst2d, source_x)


def _post_kernel(x_ref, agg_ref, wcomb_ref, whh_ref, bih_ref, bhh_ref, out_ref):
    agg = agg_ref[...]
    gates = jax.lax.dot_general(
        x_ref[...], wcomb_ref[...], (((1,), (1,)), ((), ())),
        preferred_element_type=jnp.float32)
    gates += jax.lax.dot_general(
        agg, whh_ref[...], (((1,), (1,)), ((), ())),
        preferred_element_type=jnp.float32)
    gates += bih_ref[...] + bhh_ref[...]
    i_g = jax.nn.sigmoid(gates[:, 0 * D_OUT:1 * D_OUT])
    f_g = jax.nn.sigmoid(gates[:, 1 * D_OUT:2 * D_OUT])
    g_g = jnp.tanh(gates[:, 2 * D_OUT:3 * D_OUT])
    o_g = jax.nn.sigmoid(gates[:, 3 * D_OUT:4 * D_OUT])
    c = f_g * agg + i_g * g_g
    h = o_g * jnp.tanh(c)
    out_ref[...] = jnp.maximum(h, 0.0)


def _post(x, agg, W_comb, W_hh, b_ih2, b_hh2):
    return pl.pallas_call(
        _post_kernel,
        grid=(_NB,),
        in_specs=[
            pl.BlockSpec((_BN, D_IN), lambda i: (i, 0)),
            pl.BlockSpec((_BN, D_OUT), lambda i: (i, 0)),
            pl.BlockSpec((G, D_IN), lambda i: (0, 0)),
            pl.BlockSpec((G, D_OUT), lambda i: (0, 0)),
            pl.BlockSpec((1, G), lambda i: (0, 0)),
            pl.BlockSpec((1, G), lambda i: (0, 0)),
        ],
        out_specs=pl.BlockSpec((_BN, D_OUT), lambda i: (i, 0)),
        out_shape=jax.ShapeDtypeStruct((N, D_OUT), jnp.float32),
    )(x, agg, W_comb, W_hh, b_ih2, b_hh2)


def kernel(x, edge_index, W_src, W_tgt, W_ih, W_hh, b_ih, b_hh):
    src2d = edge_index[0].reshape(_EB, 1, _BE)
    dst2d = edge_index[1].reshape(_EB, 1, _BE)
    source_x, W_comb = _pre(x, W_src, W_ih, W_tgt)
    agg = _aggregate(src2d, dst2d, source_x)
    return _post(x, agg, W_comb, W_hh,
                 b_ih.reshape(1, G), b_hh.reshape(1, G))
